# baseline (speedup 1.0000x reference)
import functools

import jax
import jax.numpy as jnp
from jax import lax
from jax.experimental import pallas as pl
from jax.experimental.pallas import tpu as pltpu
from jax.experimental.pallas import tpu_sc as plsc

SEQ_LEN = 4096
EMBED_DIM = 1024
BATCH = 4
TOTAL = BATCH * SEQ_LEN

NUM_CORES = 2
NUM_SUBCORES = 16
NUM_WORKERS = NUM_CORES * NUM_SUBCORES

ROWS_PER_WORKER = TOTAL // NUM_WORKERS
WORKERS_PER_BATCH = SEQ_LEN // ROWS_PER_WORKER
CHUNK = 32
N_CHUNKS = ROWS_PER_WORKER // CHUNK
NBUF = 3

_mesh = plsc.VectorSubcoreMesh(core_axis_name="c", subcore_axis_name="s")


@functools.partial(
    pl.kernel,
    mesh=_mesh,
    out_type=jax.ShapeDtypeStruct((TOTAL, EMBED_DIM), jnp.float32),
    scratch_types=[
        pltpu.VMEM((ROWS_PER_WORKER,), jnp.int32),
        pltpu.VMEM((NBUF, CHUNK, EMBED_DIM), jnp.float32),
        pltpu.SemaphoreType.DMA,
        pltpu.SemaphoreType.DMA,
        pltpu.SemaphoreType.DMA,
        pltpu.SemaphoreType.DMA,
        pltpu.SemaphoreType.DMA,
        pltpu.SemaphoreType.DMA,
    ],
)
def _gather_kernel(table_hbm, idx_hbm, out_hbm, idx_v, bufs,
                   gsem0, gsem1, gsem2, ssem0, ssem1, ssem2):
    gsems = (gsem0, gsem1, gsem2)
    ssems = (ssem0, ssem1, ssem2)
    wid = lax.axis_index("s") * NUM_CORES + lax.axis_index("c")
    base = wid * ROWS_PER_WORKER
    b = wid // WORKERS_PER_BATCH
    col = (wid % WORKERS_PER_BATCH) * ROWS_PER_WORKER
    pltpu.sync_copy(idx_hbm.at[b, pl.ds(col, ROWS_PER_WORKER)], idx_v)

    def fire_gather(c):
        return pltpu.async_copy(
            table_hbm.at[idx_v.at[pl.ds(c * CHUNK, CHUNK)]],
            bufs.at[c % NBUF], gsems[c % NBUF])

    def fire_scatter(c):
        return pltpu.async_copy(
            bufs.at[c % NBUF],
            out_hbm.at[pl.ds(base + c * CHUNK, CHUNK)], ssems[c % NBUF])

    gathers = [None] * N_CHUNKS
    scatters = [None] * N_CHUNKS
    for c in range(NBUF - 1):
        gathers[c] = fire_gather(c)
    for c in range(N_CHUNKS):
        if c + 1 < N_CHUNKS:
            if c >= NBUF - 1:
                scatters[c + 1 - NBUF].wait()
            gathers[c + 1] = fire_gather(c + 1)
        gathers[c].wait()
        scatters[c] = fire_scatter(c)
    for c in range(N_CHUNKS - NBUF, N_CHUNKS):
        scatters[c].wait()


def kernel(input_positions, position_embeddings):
    out = _gather_kernel(position_embeddings,
                         input_positions.astype(jnp.int32))
    return jnp.reshape(out, (BATCH, SEQ_LEN, EMBED_DIM))


# speedup vs baseline: 1.0120x; 1.0120x over previous
import functools

import jax
import jax.numpy as jnp
from jax import lax
from jax.experimental import pallas as pl
from jax.experimental.pallas import tpu as pltpu
from jax.experimental.pallas import tpu_sc as plsc

SEQ_LEN = 4096
EMBED_DIM = 1024
BATCH = 4
TOTAL = BATCH * SEQ_LEN

NUM_CORES = 2
NUM_SUBCORES = 16
NUM_WORKERS = NUM_CORES * NUM_SUBCORES

ROWS_PER_WORKER = TOTAL // NUM_WORKERS
WORKERS_PER_BATCH = SEQ_LEN // ROWS_PER_WORKER
CHUNK = 32
N_CHUNKS = ROWS_PER_WORKER // CHUNK
NBUF = 3

_mesh = plsc.VectorSubcoreMesh(core_axis_name="c", subcore_axis_name="s")


@functools.partial(
    pl.kernel,
    mesh=_mesh,
    out_type=jax.ShapeDtypeStruct((TOTAL, EMBED_DIM), jnp.float32),
    scratch_types=[
        pltpu.VMEM((ROWS_PER_WORKER,), jnp.int32),
        pltpu.VMEM((NBUF, CHUNK, EMBED_DIM), jnp.float32),
        pltpu.SemaphoreType.DMA,
        pltpu.SemaphoreType.DMA,
        pltpu.SemaphoreType.DMA,
        pltpu.SemaphoreType.DMA,
        pltpu.SemaphoreType.DMA,
        pltpu.SemaphoreType.DMA,
    ],
)
def _gather_kernel(table_hbm, idx_hbm, out_hbm, idx_v, bufs,
                   gsem0, gsem1, gsem2, ssem0, ssem1, ssem2):
    gsems = (gsem0, gsem1, gsem2)
    ssems = (ssem0, ssem1, ssem2)
    wid = lax.axis_index("s") * NUM_CORES + lax.axis_index("c")
    base = wid * ROWS_PER_WORKER
    b = wid // WORKERS_PER_BATCH
    col = (wid % WORKERS_PER_BATCH) * ROWS_PER_WORKER
    pltpu.sync_copy(idx_hbm.at[b, pl.ds(col, ROWS_PER_WORKER)], idx_v)

    def fire_gather(c):
        return pltpu.async_copy(
            table_hbm.at[idx_v.at[pl.ds(c * CHUNK, CHUNK)]],
            bufs.at[c % NBUF], gsems[c % NBUF])

    def fire_scatter(c):
        return pltpu.async_copy(
            bufs.at[c % NBUF],
            out_hbm.at[pl.ds(base + c * CHUNK, CHUNK)], ssems[c % NBUF])

    gathers = [None] * N_CHUNKS
    scatters = [None] * N_CHUNKS
    gathers[0] = fire_gather(0)
    for c in range(N_CHUNKS):
        if c + 1 < N_CHUNKS:
            if c >= NBUF - 1:
                scatters[c + 1 - NBUF].wait()
            gathers[c + 1] = fire_gather(c + 1)
        gathers[c].wait()
        scatters[c] = fire_scatter(c)
    for c in range(N_CHUNKS - NBUF, N_CHUNKS):
        scatters[c].wait()


def kernel(input_positions, position_embeddings):
    out = _gather_kernel(position_embeddings,
                         input_positions.astype(jnp.int32))
    return jnp.reshape(out, (BATCH, SEQ_LEN, EMBED_DIM))


# speedup vs baseline: 1.0227x; 1.0105x over previous
import functools

import jax
import jax.numpy as jnp
from jax import lax
from jax.experimental import pallas as pl
from jax.experimental.pallas import tpu as pltpu
from jax.experimental.pallas import tpu_sc as plsc

SEQ_LEN = 4096
EMBED_DIM = 1024
BATCH = 4
TOTAL = BATCH * SEQ_LEN

NUM_CORES = 2
NUM_SUBCORES = 16
NUM_WORKERS = NUM_CORES * NUM_SUBCORES

ROWS_PER_WORKER = TOTAL // NUM_WORKERS
WORKERS_PER_BATCH = SEQ_LEN // ROWS_PER_WORKER
CHUNK = 16
N_CHUNKS = ROWS_PER_WORKER // CHUNK
NBUF = 4

_mesh = plsc.VectorSubcoreMesh(core_axis_name="c", subcore_axis_name="s")


@functools.partial(
    pl.kernel,
    mesh=_mesh,
    out_type=jax.ShapeDtypeStruct((TOTAL, EMBED_DIM), jnp.float32),
    scratch_types=[
        pltpu.VMEM((ROWS_PER_WORKER,), jnp.int32),
        pltpu.VMEM((NBUF, CHUNK, EMBED_DIM), jnp.float32),
        pltpu.SemaphoreType.DMA,
        pltpu.SemaphoreType.DMA,
        pltpu.SemaphoreType.DMA,
        pltpu.SemaphoreType.DMA,
        pltpu.SemaphoreType.DMA,
        pltpu.SemaphoreType.DMA,
        pltpu.SemaphoreType.DMA,
        pltpu.SemaphoreType.DMA,
    ],
)
def _gather_kernel(table_hbm, idx_hbm, out_hbm, idx_v, bufs,
                   gsem0, gsem1, gsem2, gsem3, ssem0, ssem1, ssem2, ssem3):
    gsems = (gsem0, gsem1, gsem2, gsem3)
    ssems = (ssem0, ssem1, ssem2, ssem3)
    wid = lax.axis_index("s") * NUM_CORES + lax.axis_index("c")
    base = wid * ROWS_PER_WORKER
    b = wid // WORKERS_PER_BATCH
    col = (wid % WORKERS_PER_BATCH) * ROWS_PER_WORKER
    pltpu.sync_copy(idx_hbm.at[b, pl.ds(col, ROWS_PER_WORKER)], idx_v)

    def fire_gather(c):
        return pltpu.async_copy(
            table_hbm.at[idx_v.at[pl.ds(c * CHUNK, CHUNK)]],
            bufs.at[c % NBUF], gsems[c % NBUF])

    def fire_scatter(c):
        return pltpu.async_copy(
            bufs.at[c % NBUF],
            out_hbm.at[pl.ds(base + c * CHUNK, CHUNK)], ssems[c % NBUF])

    gathers = [None] * N_CHUNKS
    scatters = [None] * N_CHUNKS
    gathers[0] = fire_gather(0)
    gathers[1] = fire_gather(1)
    for c in range(N_CHUNKS):
        if c + 2 < N_CHUNKS:
            if c >= NBUF - 2:
                scatters[c + 2 - NBUF].wait()
            gathers[c + 2] = fire_gather(c + 2)
        gathers[c].wait()
        scatters[c] = fire_scatter(c)
    for c in range(N_CHUNKS - NBUF, N_CHUNKS):
        scatters[c].wait()


def kernel(input_positions, position_embeddings):
    out = _gather_kernel(position_embeddings,
                         input_positions.astype(jnp.int32))
    return jnp.reshape(out, (BATCH, SEQ_LEN, EMBED_DIM))


# speedup vs baseline: 1.0292x; 1.0064x over previous
import functools

import jax
import jax.numpy as jnp
from jax import lax
from jax.experimental import pallas as pl
from jax.experimental.pallas import tpu as pltpu
from jax.experimental.pallas import tpu_sc as plsc

SEQ_LEN = 4096
EMBED_DIM = 1024
BATCH = 4
TOTAL = BATCH * SEQ_LEN

NUM_CORES = 2
NUM_SUBCORES = 16
NUM_WORKERS = NUM_CORES * NUM_SUBCORES

ROWS_PER_WORKER = TOTAL // NUM_WORKERS
WORKERS_PER_BATCH = SEQ_LEN // ROWS_PER_WORKER
CHUNK = 32
N_CHUNKS = ROWS_PER_WORKER // CHUNK
NBUF = 2

_mesh = plsc.VectorSubcoreMesh(core_axis_name="c", subcore_axis_name="s")


@functools.partial(
    pl.kernel,
    mesh=_mesh,
    out_type=jax.ShapeDtypeStruct((TOTAL, EMBED_DIM), jnp.float32),
    scratch_types=[
        pltpu.VMEM((ROWS_PER_WORKER,), jnp.int32),
        pltpu.VMEM((NBUF, CHUNK, EMBED_DIM), jnp.float32),
        pltpu.SemaphoreType.DMA,
        pltpu.SemaphoreType.DMA,
    ],
)
def _gather_kernel(table_hbm, idx_hbm, out_hbm, idx_v, bufs, gsem, ssem):
    wid = lax.axis_index("s") * NUM_CORES + lax.axis_index("c")
    base = wid * ROWS_PER_WORKER
    b = wid // WORKERS_PER_BATCH
    col = (wid % WORKERS_PER_BATCH) * ROWS_PER_WORKER
    pltpu.sync_copy(idx_hbm.at[b, pl.ds(col, ROWS_PER_WORKER)], idx_v)
    gathers = [None] * NBUF
    scatters = [None] * NBUF
    for j in range(NBUF - 1):
        gathers[j] = pltpu.async_copy(
            table_hbm.at[idx_v.at[pl.ds(j * CHUNK, CHUNK)]], bufs.at[j], gsem)
    for j in range(N_CHUNKS):
        cur = j % NBUF
        nxt = (j + NBUF - 1) % NBUF
        if j + NBUF - 1 < N_CHUNKS:
            if scatters[nxt] is not None:
                scatters[nxt].wait()
            gathers[nxt] = pltpu.async_copy(
                table_hbm.at[idx_v.at[pl.ds((j + NBUF - 1) * CHUNK, CHUNK)]],
                bufs.at[nxt], gsem)
        gathers[cur].wait()
        scatters[cur] = pltpu.async_copy(
            bufs.at[cur], out_hbm.at[pl.ds(base + j * CHUNK, CHUNK)], ssem)
    for j in range(NBUF):
        scatters[(N_CHUNKS - NBUF + j) % NBUF].wait()


def kernel(input_positions, position_embeddings):
    out = _gather_kernel(position_embeddings,
                         input_positions.astype(jnp.int32))
    return jnp.reshape(out, (BATCH, SEQ_LEN, EMBED_DIM))
